# Initial kernel scaffold; baseline (speedup 1.0000x reference)
import jax
import jax.numpy as jnp
from jax.experimental import pallas as pl


def kernel(x, W_trans, b_trans, w_fit):
    raise NotImplementedError("write your pallas kernel here")


# speedup vs baseline: 2.3165x; 2.3165x over previous
import functools

import jax
import jax.numpy as jnp
from jax.experimental import pallas as pl
from jax.experimental.pallas import tpu as pltpu

_BRANCH = 16
_BEAM = 32
_D = 64
_B = 1024
_BB = 128

_NEG = -1e30


def _tc_body(x_ref, w_ref, b_ref, wf_ref, y_ref,
             s1_ref, fit2_ref, i2_ref, s2_ref, fit3_ref, vals_ref, i3_ref,
             e_ref):
    BB = _BB
    wfT = wf_ref[:, :].reshape(1, _D)

    def fit_row(t):
        return jax.lax.dot_general(
            wfT, t, (((1,), (0,)), ((), ())),
            precision=jax.lax.Precision.DEFAULT,
            preferred_element_type=jnp.float32)

    for j in range(_BRANCH):
        s1_ref[j] = jnp.tanh(x_ref[:, :] * w_ref[j] + b_ref[j])

    def fit2_step(j, _):
        wc = w_ref[j]
        bc = b_ref[j]
        rows = []
        for i in range(_BRANCH):
            t = jnp.tanh(s1_ref[i] * wc + bc)
            rows.append(fit_row(t))
        fit2_ref[pl.ds(j * _BRANCH, _BRANCH), :] = jnp.concatenate(rows, axis=0)
        return 0

    jax.lax.fori_loop(0, _BRANCH, fit2_step, 0)

    iota2 = jax.lax.broadcasted_iota(jnp.int32, (_BRANCH * _BRANCH, BB), 0)

    def top2_step(k, _):
        v = fit2_ref[:, :]
        m = jnp.max(v, axis=0, keepdims=True)
        idx = jnp.min(jnp.where(v == m, iota2, jnp.int32(1 << 30)),
                      axis=0, keepdims=True)
        i2_ref[k] = idx
        fit2_ref[:, :] = jnp.where(iota2 == idx, _NEG, v)
        return 0

    jax.lax.fori_loop(0, _BEAM, top2_step, 0)

    def build2_step(k, _):
        c = i2_ref[k]
        ip = jnp.remainder(c, _BRANCH)
        jb = c // _BRANCH
        ps = s1_ref[0]
        for i in range(1, _BRANCH):
            ps = jnp.where(ip == i, s1_ref[i], ps)
        wc = jnp.broadcast_to(w_ref[0], (_D, BB))
        bc = jnp.broadcast_to(b_ref[0], (_D, BB))
        for j in range(1, _BRANCH):
            wc = jnp.where(jb == j, w_ref[j], wc)
            bc = jnp.where(jb == j, b_ref[j], bc)
        s2_ref[k] = jnp.tanh(ps * wc + bc)
        return 0

    jax.lax.fori_loop(0, _BEAM, build2_step, 0)

    def fit3_step(j, _):
        wc = w_ref[j]
        bc = b_ref[j]
        rows = []
        for i in range(_BEAM):
            t = jnp.tanh(s2_ref[i] * wc + bc)
            rows.append(fit_row(t))
        fit3_ref[pl.ds(j * _BEAM, _BEAM), :] = jnp.concatenate(rows, axis=0)
        return 0

    jax.lax.fori_loop(0, _BRANCH, fit3_step, 0)

    iota3 = jax.lax.broadcasted_iota(jnp.int32, (_BRANCH * _BEAM, BB), 0)

    def top3_step(k, _):
        v = fit3_ref[:, :]
        m = jnp.max(v, axis=0, keepdims=True)
        idx = jnp.min(jnp.where(v == m, iota3, jnp.int32(1 << 30)),
                      axis=0, keepdims=True)
        i3_ref[k] = idx
        vals_ref[k] = m
        fit3_ref[:, :] = jnp.where(iota3 == idx, _NEG, v)
        return 0

    jax.lax.fori_loop(0, _BEAM, top3_step, 0)

    m = vals_ref[0]
    for k in range(1, _BEAM):
        m = jnp.maximum(m, vals_ref[k])
    z = jnp.zeros((1, BB), jnp.float32)
    for k in range(_BEAM):
        e = jnp.exp(vals_ref[k] - m)
        e_ref[k] = e
        z = z + e

    def out_step(k, acc):
        c = i3_ref[k]
        ip = jnp.remainder(c, _BEAM)
        jb = c // _BEAM
        ps = s2_ref[0]
        for i in range(1, _BEAM):
            ps = jnp.where(ip == i, s2_ref[i], ps)
        wc = jnp.broadcast_to(w_ref[0], (_D, BB))
        bc = jnp.broadcast_to(b_ref[0], (_D, BB))
        for j in range(1, _BRANCH):
            wc = jnp.where(jb == j, w_ref[j], wc)
            bc = jnp.where(jb == j, b_ref[j], bc)
        st = jnp.tanh(ps * wc + bc)
        return acc + st * (e_ref[k] / z)

    acc = jax.lax.fori_loop(0, _BEAM, out_step,
                            jnp.zeros((_D, BB), jnp.float32))
    y_ref[:, :] = acc


@jax.jit
def kernel(x, W_trans, b_trans, w_fit):
    xT = x.T
    wcol = W_trans[:, :, None]
    bcol = b_trans[:, :, None]

    grid = (_B // _BB,)
    yT = pl.pallas_call(
        _tc_body,
        grid=grid,
        in_specs=[
            pl.BlockSpec((_D, _BB), lambda i: (0, i)),
            pl.BlockSpec((_BRANCH, _D, 1), lambda i: (0, 0, 0)),
            pl.BlockSpec((_BRANCH, _D, 1), lambda i: (0, 0, 0)),
            pl.BlockSpec((_D, 1), lambda i: (0, 0)),
        ],
        out_specs=pl.BlockSpec((_D, _BB), lambda i: (0, i)),
        out_shape=jax.ShapeDtypeStruct((_D, _B), jnp.float32),
        scratch_shapes=[
            pltpu.VMEM((_BRANCH, _D, _BB), jnp.float32),
            pltpu.VMEM((_BRANCH * _BRANCH, _BB), jnp.float32),
            pltpu.VMEM((_BEAM, 1, _BB), jnp.int32),
            pltpu.VMEM((_BEAM, _D, _BB), jnp.float32),
            pltpu.VMEM((_BRANCH * _BEAM, _BB), jnp.float32),
            pltpu.VMEM((_BEAM, 1, _BB), jnp.float32),
            pltpu.VMEM((_BEAM, 1, _BB), jnp.int32),
            pltpu.VMEM((_BEAM, 1, _BB), jnp.float32),
        ],
    )(xT, wcol, bcol, w_fit)
    return yT.T
